# baseline (speedup 1.0000x reference)
import functools

import jax
import jax.numpy as jnp
from jax import lax
from jax.experimental import pallas as pl
from jax.experimental.pallas import tpu as pltpu
from jax.experimental.pallas import tpu_sc as plsc

_LANES = 16
_NW = 32
_SUB = 100


def _build(vocab, maxlen, embed, batch):
  nrows = batch * maxlen
  bpw = nrows // _NW
  nsub = bpw // _SUB
  chunk = maxlen
  sub_per_ch = chunk // _SUB
  nch = bpw // chunk
  evec = embed // _LANES

  mesh = plsc.VectorSubcoreMesh(core_axis_name="c", subcore_axis_name="s")
  nc = 2

  @functools.partial(
      pl.kernel,
      mesh=mesh,
      out_type=jax.ShapeDtypeStruct((nrows, embed), jnp.float32),
      compiler_params=pltpu.CompilerParams(use_tc_tiling_on_sc=False),
      scratch_types=[
          pltpu.VMEM((nsub, _SUB), jnp.int32),
          pltpu.VMEM((chunk, embed), jnp.float32),
          pltpu.VMEM((maxlen, embed), jnp.float32),
          pltpu.SemaphoreType.DMA,
      ],
  )
  def emb(x_hbm, tok_hbm, pos_hbm, out_hbm, idx_v, rows_v, pos_v, gsem):
    wid = lax.axis_index("s") * nc + lax.axis_index("c")
    base_sub = wid * nsub
    base_row = wid * bpw
    pltpu.sync_copy(x_hbm.at[pl.ds(base_sub, nsub)], idx_v)
    pltpu.sync_copy(pos_hbm, pos_v)

    def chunk_body(i, carry):
      handles = []
      for j in range(sub_per_ch):
        handles.append(pltpu.async_copy(
            tok_hbm.at[idx_v.at[i * sub_per_ch + j]],
            rows_v.at[pl.ds(j * _SUB, _SUB)],
            gsem,
        ))
      for h in handles:
        h.wait()

      def add_body(r, acc):
        for c in range(evec):
          sl = pl.ds(c * _LANES, _LANES)
          rows_v[r, sl] = rows_v[r, sl] + pos_v[r, sl]
        return acc
      lax.fori_loop(0, chunk, add_body, 0)

      pltpu.sync_copy(rows_v, out_hbm.at[pl.ds(base_row + i * chunk, chunk)])
      return carry

    lax.fori_loop(0, nch, chunk_body, 0)

  return emb


def kernel(x, token_table, pos_table):
  batch, maxlen = x.shape
  vocab, embed = token_table.shape
  xf = x.astype(jnp.int32).reshape(_NW * (batch * maxlen // _NW // _SUB), _SUB)
  emb = _build(vocab, maxlen, embed, batch)
  out = emb(xf, token_table, pos_table)
  return out.reshape(batch, maxlen, embed)


# speedup vs baseline: 1.0996x; 1.0996x over previous
import functools

import jax
import jax.numpy as jnp
from jax import lax
from jax.experimental import pallas as pl
from jax.experimental.pallas import tpu as pltpu
from jax.experimental.pallas import tpu_sc as plsc

_LANES = 16
_NW = 32
_SUB = 100


def _build(vocab, maxlen, embed, batch):
  nrows = batch * maxlen
  bpw = nrows // _NW
  nsub = bpw // _SUB
  chunk = 2 * maxlen
  sub_per_ch = chunk // _SUB
  nch = bpw // chunk
  evec = embed // _LANES

  mesh = plsc.VectorSubcoreMesh(core_axis_name="c", subcore_axis_name="s")
  nc = 2

  @functools.partial(
      pl.kernel,
      mesh=mesh,
      out_type=jax.ShapeDtypeStruct((nrows, embed), jnp.float32),
      compiler_params=pltpu.CompilerParams(use_tc_tiling_on_sc=False),
      scratch_types=[
          pltpu.VMEM((nsub, _SUB), jnp.int32),
          pltpu.VMEM((chunk, embed), jnp.float32),
          pltpu.VMEM((chunk, embed), jnp.float32),
          pltpu.VMEM((maxlen, embed), jnp.float32),
          pltpu.SemaphoreType.DMA,
          pltpu.SemaphoreType.DMA,
      ],
  )
  def emb(x_hbm, tok_hbm, pos_hbm, out_hbm, idx_v, buf0, buf1, pos_v, gsem,
          osem):
    wid = lax.axis_index("s") * nc + lax.axis_index("c")
    base_sub = wid * nsub
    base_row = wid * bpw
    bufs = (buf0, buf1)

    pltpu.sync_copy(x_hbm.at[pl.ds(base_sub, nsub)], idx_v)
    pltpu.sync_copy(pos_hbm, pos_v)

    def gather_descs(i, buf):
      return [
          pltpu.make_async_copy(
              tok_hbm.at[idx_v.at[i * sub_per_ch + j]],
              buf.at[pl.ds(j * _SUB, _SUB)],
              gsem,
          )
          for j in range(sub_per_ch)
      ]

    def store_desc(i, buf):
      return pltpu.make_async_copy(
          buf, out_hbm.at[pl.ds(base_row + i * chunk, chunk)], osem)

    for d in gather_descs(0, bufs[0]):
      d.start()

    def pair_body(g, carry):
      for b in range(2):
        i = g * 2 + b
        buf, other = bufs[b], bufs[b ^ 1]

        for d in gather_descs(i, buf):
          d.wait()

        def add_body(r, acc):
          for c in range(evec):
            sl = pl.ds(c * _LANES, _LANES)
            p = pos_v[r, sl]
            plsc.addupdate(buf.at[r, sl], p)
            plsc.addupdate(buf.at[maxlen + r, sl], p)
          return acc
        lax.fori_loop(0, maxlen, add_body, 0)

        @pl.when(i >= 1)
        def _():
          store_desc(i - 1, other).wait()

        store_desc(i, buf).start()

        @pl.when(i + 1 < nch)
        def _():
          for d in gather_descs(i + 1, other):
            d.start()
      return carry

    lax.fori_loop(0, nch // 2, pair_body, 0)
    store_desc(nch - 1, bufs[1]).wait()

  return emb


def kernel(x, token_table, pos_table):
  batch, maxlen = x.shape
  vocab, embed = token_table.shape
  xf = x.astype(jnp.int32).reshape(_NW * (batch * maxlen // _NW // _SUB), _SUB)
  emb = _build(vocab, maxlen, embed, batch)
  out = emb(xf, token_table, pos_table)
  return out.reshape(batch, maxlen, embed)
